# baseline (speedup 1.0000x reference)
import functools

import jax
import jax.numpy as jnp
from jax import lax
from jax.experimental import pallas as pl
from jax.experimental.pallas import tpu as pltpu
from jax.experimental.pallas import tpu_sc as plsc

_W = 16
_L = 16


@functools.lru_cache(maxsize=None)
def _make_lookup(vocab: int, embed: int, batch: int):
  info = plsc.get_sparse_core_info()
  nc, ns = info.num_cores, info.num_subcores
  jh = embed // (2 * nc)
  bpw = batch // (ns // 2)
  waves = bpw // _W
  assert waves % 2 == 0
  mesh = plsc.VectorSubcoreMesh(core_axis_name="c", subcore_axis_name="s")

  @functools.partial(
      pl.kernel,
      mesh=mesh,
      out_type=jax.ShapeDtypeStruct((embed, batch), jnp.float32),
      scratch_types=[
          pltpu.VMEM((bpw,), jnp.int32),
          pltpu.VMEM((jh, bpw), jnp.float32),
          pltpu.SemaphoreType.DMA,
          pltpu.SemaphoreType.DMA,
      ]
      + [pltpu.VMEM((jh, 128), jnp.float32) for _ in range(2 * _W)],
      compiler_params=pltpu.CompilerParams(needs_layout_passes=False),
  )
  def lookup(emb_hbm, idx_hbm, out_hbm, idx_s, t_v, sem_i, sem, *slabs):
    iota = lax.iota(jnp.int32, _L)
    s = lax.axis_index("s")
    jb = pl.multiple_of(lax.axis_index("c") * 2 * jh + (s % 2) * jh, 8)
    base = (s // 2) * bpw
    pltpu.async_copy(idx_hbm.at[pl.ds(base, bpw)], idx_s, sem_i).wait()

    def fire_wave(g, bank):
      rv = idx_s[pl.ds(g * _W, _W)]
      cbv = (rv // 128) * 128
      for ss in range(_W):
        cb = pl.multiple_of(cbv[ss], 128)
        pltpu.async_copy(
            emb_hbm.at[pl.ds(jb, jh), pl.ds(cb, 128)],
            slabs[bank * _W + ss], sem,
        )

    def drain_wave():
      cp = pltpu.make_async_copy(
          emb_hbm.at[pl.ds(0, jh), pl.ds(0, 128)], slabs[0], sem
      )
      for _ in range(_W):
        cp.wait()

    def extract_wave(g, bank):
      rv = idx_s[pl.ds(g * _W, _W)]
      lanev = rv & 127
      rows = iota & (jh - 1)
      for ss in range(_W):
        lane = jnp.full((_L,), lanev[ss], jnp.int32)
        col = plsc.load_gather(slabs[bank * _W + ss], [rows, lane])
        bcol = jnp.full((_L,), g * _W + ss, jnp.int32)
        plsc.store_scatter(t_v, [rows, bcol], col)

    fire_wave(0, 0)

    def step(g2, carry):
      g = g2 * 2
      for half in range(2):

        @pl.when(g + half + 1 < waves)
        def _():
          fire_wave(g + half + 1, 1 - half)

        drain_wave()
        extract_wave(g + half, half)
      return carry

    lax.fori_loop(0, waves // 2, step, 0)
    pltpu.sync_copy(t_v, out_hbm.at[pl.ds(jb, jh), pl.ds(base, bpw)])

  return lookup


def kernel(inputs, embeddings):
  batch, = inputs.shape
  vocab, embed = embeddings.shape
  idx = inputs.astype(jnp.int32)
  out_t = _make_lookup(vocab, embed, batch)(embeddings.T, idx)
  return out_t.T


# speedup vs baseline: 1.2431x; 1.2431x over previous
import functools

import jax
import jax.numpy as jnp
from jax import lax
from jax.experimental import pallas as pl
from jax.experimental.pallas import tpu as pltpu
from jax.experimental.pallas import tpu_sc as plsc

_W = 8
_L = 16


@functools.lru_cache(maxsize=None)
def _make_lookup(vocab: int, embed: int, batch: int):
  info = plsc.get_sparse_core_info()
  nc, ns = info.num_cores, info.num_subcores
  nw = nc * ns
  bpw = batch // nw
  waves = bpw // _W
  assert waves % 2 == 0 and embed == 2 * _L
  mesh = plsc.VectorSubcoreMesh(core_axis_name="c", subcore_axis_name="s")

  @functools.partial(
      pl.kernel,
      mesh=mesh,
      out_type=jax.ShapeDtypeStruct((embed, batch), jnp.float32),
      scratch_types=[
          pltpu.VMEM((bpw + _L,), jnp.int32),
          pltpu.VMEM((embed, bpw), jnp.float32),
          pltpu.SemaphoreType.DMA,
          pltpu.SemaphoreType.DMA,
      ]
      + [pltpu.VMEM((embed, 128), jnp.float32) for _ in range(2 * _W)],
      compiler_params=pltpu.CompilerParams(needs_layout_passes=False),
  )
  def lookup(emb_hbm, idx_hbm, out_hbm, idx_s, t_v, sem_i, sem, *slabs):
    iota = lax.iota(jnp.int32, _L)
    wid = lax.axis_index("s") * nc + lax.axis_index("c")
    base = wid * bpw
    pltpu.async_copy(
        idx_hbm.at[pl.ds(base, bpw)], idx_s.at[pl.ds(0, bpw)], sem_i
    ).wait()

    def fire_wave(g, bank):
      rv = idx_s[pl.ds(g * _W, _L)]
      cbv = (rv // 128) * 128
      for ss in range(_W):
        cb = pl.multiple_of(cbv[ss], 128)
        pltpu.async_copy(
            emb_hbm.at[:, pl.ds(cb, 128)], slabs[bank * _W + ss], sem
        )

    def drain_wave():
      cp = pltpu.make_async_copy(
          emb_hbm.at[:, pl.ds(0, 128)], slabs[0], sem
      )
      for _ in range(_W):
        cp.wait()

    def extract_wave(g, bank):
      rv = idx_s[pl.ds(g * _W, _L)]
      lanev = rv & 127
      for ss in range(_W):
        lane = jnp.full((_L,), lanev[ss], jnp.int32)
        bcol = jnp.full((_L,), g * _W + ss, jnp.int32)
        for h in range(2):
          col = plsc.load_gather(
              slabs[bank * _W + ss], [h * _L + iota, lane]
          )
          plsc.store_scatter(t_v, [h * _L + iota, bcol], col)

    fire_wave(0, 0)

    def step(g2, carry):
      g = g2 * 2
      for half in range(2):

        @pl.when(g + half + 1 < waves)
        def _():
          fire_wave(g + half + 1, 1 - half)

        drain_wave()
        extract_wave(g + half, half)
      return carry

    lax.fori_loop(0, waves // 2, step, 0)
    pltpu.sync_copy(t_v, out_hbm.at[:, pl.ds(base, bpw)])

  return lookup


def kernel(inputs, embeddings):
  batch, = inputs.shape
  vocab, embed = embeddings.shape
  idx = inputs.astype(jnp.int32)
  out_t = _make_lookup(vocab, embed, batch)(embeddings.T, idx)
  return out_t.T
